# Initial kernel scaffold; baseline (speedup 1.0000x reference)
import jax
import jax.numpy as jnp
from jax.experimental import pallas as pl


def kernel(logits):
    raise NotImplementedError("write your pallas kernel here")


# speedup vs baseline: 17.1114x; 17.1114x over previous
import functools

import jax
import jax.numpy as jnp
from jax import lax
from jax.experimental import pallas as pl

_K = 64
_ROWS = 8
_M = 32768


def _topk_mask_body(x_ref, o_ref):
    x = x_ref[0]
    i = lax.bitcast_convert_type(x, jnp.int32)
    m = i ^ ((i >> 31) & jnp.int32(0x7FFFFFFF))

    kf = jnp.float32(_K)

    def cond(carry):
        b, p, cnt = carry
        return (b >= 0) & jnp.logical_not(jnp.all(cnt == kf))

    def body(carry):
        b, p, cnt = carry
        t = p ^ (jnp.int32(1) << b)
        c = jnp.sum((m >= t).astype(jnp.float32), axis=-1, keepdims=True)
        take = c >= kf
        p = jnp.where(take, t, p)
        cnt = jnp.where(take, c, cnt)
        return b - 1, p, cnt

    p0 = jnp.full((_ROWS, 1), jnp.int32(-2147483648))
    cnt0 = jnp.full((_ROWS, 1), jnp.float32(_M))
    _, p, cnt = lax.while_loop(cond, body, (jnp.int32(31), p0, cnt0))

    all_resolved = jnp.all(cnt == kf)
    eq = m == p
    cnt_eq = jnp.sum(eq.astype(jnp.float32), axis=-1, keepdims=True)
    r = kf - (cnt - cnt_eq)
    idx = lax.broadcasted_iota(jnp.int32, (_ROWS, _M), 1)

    def cond2(carry):
        b2, _ = carry
        return (b2 >= 0) & jnp.logical_not(all_resolved)

    def body2(carry):
        b2, p2 = carry
        t2 = p2 | (jnp.int32(1) << b2)
        f = jnp.sum((eq & (idx < t2)).astype(jnp.float32), axis=-1,
                    keepdims=True)
        return b2 - 1, jnp.where(f < r, t2, p2)

    _, p2 = lax.while_loop(cond2, body2,
                           (jnp.int32(14), jnp.zeros((_ROWS, 1), jnp.int32)))
    j = jnp.where(cnt == kf, jnp.int32(_M - 1), p2)
    mask = (m > p) | (eq & (idx <= j))

    kept = jnp.where(mask, x, jnp.float32(0.0))
    s = jnp.sum(kept, axis=-1, keepdims=True) + jnp.float32(1e-12)
    o_ref[0] = kept * (jnp.float32(1.0) / s)


def kernel(logits):
    C, L, M = logits.shape
    grid = (C * L) // _ROWS
    x = logits.reshape(grid, _ROWS, M)
    out = pl.pallas_call(
        _topk_mask_body,
        grid=(grid,),
        in_specs=[pl.BlockSpec((1, _ROWS, M), lambda g: (g, 0, 0))],
        out_specs=pl.BlockSpec((1, _ROWS, M), lambda g: (g, 0, 0)),
        out_shape=jax.ShapeDtypeStruct((grid, _ROWS, M), jnp.float32),
    )(x)
    return out.reshape(C, L, M)


# speedup vs baseline: 24.4964x; 1.4316x over previous
import functools

import jax
import jax.numpy as jnp
from jax import lax
from jax.experimental import pallas as pl

_K = 64
_ROWS = 8
_M = 32768
_CHUNKS = 64


def _topk_mask_body(x_ref, o_ref):
    x = x_ref[0]
    i = lax.bitcast_convert_type(x, jnp.int32)
    m = i ^ ((i >> 31) & jnp.int32(0x7FFFFFFF))

    kf = jnp.float32(_K)

    cmax = jnp.max(m.reshape(_ROWS, _CHUNKS, _M // _CHUNKS), axis=-1)
    lo0 = jnp.min(cmax, axis=-1, keepdims=True)
    hi0 = jnp.max(cmax, axis=-1, keepdims=True) + jnp.int32(1)
    cnt0 = jnp.sum((m >= lo0).astype(jnp.float32), axis=-1, keepdims=True)

    def cond(carry):
        it, lo, hi, cnt = carry
        return (it < 34) & jnp.logical_not(
            jnp.all((cnt == kf) | (hi - lo == 1)))

    def body(carry):
        it, lo, hi, cnt = carry
        mid = (lo >> 1) + (hi >> 1) + (lo & hi & 1)
        c = jnp.sum((m >= mid).astype(jnp.float32), axis=-1, keepdims=True)
        take = c >= kf
        lo = jnp.where(take, mid, lo)
        cnt = jnp.where(take, c, cnt)
        hi = jnp.where(take, hi, mid)
        return it + 1, lo, hi, cnt

    _, p, _, cnt = lax.while_loop(cond, body, (jnp.int32(0), lo0, hi0, cnt0))

    all_resolved = jnp.all(cnt == kf)
    eq = m == p
    cnt_eq = jnp.sum(eq.astype(jnp.float32), axis=-1, keepdims=True)
    r = kf - (cnt - cnt_eq)
    idx = lax.broadcasted_iota(jnp.int32, (_ROWS, _M), 1)

    def cond2(carry):
        b2, _ = carry
        return (b2 >= 0) & jnp.logical_not(all_resolved)

    def body2(carry):
        b2, p2 = carry
        t2 = p2 | (jnp.int32(1) << b2)
        f = jnp.sum((eq & (idx < t2)).astype(jnp.float32), axis=-1,
                    keepdims=True)
        return b2 - 1, jnp.where(f < r, t2, p2)

    _, p2 = lax.while_loop(cond2, body2,
                           (jnp.int32(14), jnp.zeros((_ROWS, 1), jnp.int32)))
    j = jnp.where(cnt == kf, jnp.int32(_M - 1), p2)
    mask = (m > p) | (eq & (idx <= j))

    kept = jnp.where(mask, x, jnp.float32(0.0))
    s = jnp.sum(kept, axis=-1, keepdims=True) + jnp.float32(1e-12)
    o_ref[0] = kept * (jnp.float32(1.0) / s)


def kernel(logits):
    C, L, M = logits.shape
    grid = (C * L) // _ROWS
    x = logits.reshape(grid, _ROWS, M)
    out = pl.pallas_call(
        _topk_mask_body,
        grid=(grid,),
        in_specs=[pl.BlockSpec((1, _ROWS, M), lambda g: (g, 0, 0))],
        out_specs=pl.BlockSpec((1, _ROWS, M), lambda g: (g, 0, 0)),
        out_shape=jax.ShapeDtypeStruct((grid, _ROWS, M), jnp.float32),
    )(x)
    return out.reshape(C, L, M)


# speedup vs baseline: 40.7437x; 1.6633x over previous
import functools

import jax
import jax.numpy as jnp
from jax import lax
from jax.experimental import pallas as pl

_K = 64
_ROWS = 16
_M = 32768
_CHUNKS = 64


def _topk_mask_body(x_ref, o_ref):
    x = x_ref[0]
    i = lax.bitcast_convert_type(x, jnp.int32)
    m = i ^ ((i >> 31) & jnp.int32(0x7FFFFFFF))

    kf = jnp.float32(_K)

    cmax = jnp.max(m.reshape(_ROWS, _CHUNKS, _M // _CHUNKS), axis=-1)
    lo0 = jnp.min(cmax, axis=-1, keepdims=True)
    hi0 = jnp.max(cmax, axis=-1, keepdims=True) + jnp.int32(1)
    cnt0 = jnp.sum((m >= lo0).astype(jnp.float32), axis=-1, keepdims=True)

    def cond(carry):
        it, lo, hi, cnt = carry
        return (it < 34) & jnp.logical_not(
            jnp.all((cnt == kf) | (hi - lo == 1)))

    def body(carry):
        it, lo, hi, cnt = carry
        mid = (lo >> 1) + (hi >> 1) + (lo & hi & 1)
        c = jnp.sum((m >= mid).astype(jnp.float32), axis=-1, keepdims=True)
        take = c >= kf
        lo = jnp.where(take, mid, lo)
        cnt = jnp.where(take, c, cnt)
        hi = jnp.where(take, hi, mid)
        return it + 1, lo, hi, cnt

    _, p, _, cnt = lax.while_loop(cond, body, (jnp.int32(0), lo0, hi0, cnt0))

    all_resolved = jnp.all(cnt == kf)
    eq = m == p
    cnt_eq = jnp.sum(eq.astype(jnp.float32), axis=-1, keepdims=True)
    r = kf - (cnt - cnt_eq)
    idx = lax.broadcasted_iota(jnp.int32, (_ROWS, _M), 1)

    def cond2(carry):
        b2, _ = carry
        return (b2 >= 0) & jnp.logical_not(all_resolved)

    def body2(carry):
        b2, p2 = carry
        t2 = p2 | (jnp.int32(1) << b2)
        f = jnp.sum((eq & (idx < t2)).astype(jnp.float32), axis=-1,
                    keepdims=True)
        return b2 - 1, jnp.where(f < r, t2, p2)

    _, p2 = lax.while_loop(cond2, body2,
                           (jnp.int32(14), jnp.zeros((_ROWS, 1), jnp.int32)))
    j = jnp.where(cnt == kf, jnp.int32(_M - 1), p2)
    mask = (m > p) | (eq & (idx <= j))

    kept = jnp.where(mask, x, jnp.float32(0.0))
    s = jnp.sum(kept, axis=-1, keepdims=True) + jnp.float32(1e-12)
    o_ref[0] = kept * (jnp.float32(1.0) / s)


def kernel(logits):
    C, L, M = logits.shape
    grid = (C * L) // _ROWS
    x = logits.reshape(grid, _ROWS, M)
    out = pl.pallas_call(
        _topk_mask_body,
        grid=(grid,),
        in_specs=[pl.BlockSpec((1, _ROWS, M), lambda g: (g, 0, 0))],
        out_specs=pl.BlockSpec((1, _ROWS, M), lambda g: (g, 0, 0)),
        out_shape=jax.ShapeDtypeStruct((grid, _ROWS, M), jnp.float32),
    )(x)
    return out.reshape(C, L, M)


# speedup vs baseline: 45.3263x; 1.1125x over previous
import functools

import jax
import jax.numpy as jnp
from jax import lax
from jax.experimental import pallas as pl

_K = 64
_ROWS = 32
_M = 32768
_CHUNKS = 64


def _topk_mask_body(x_ref, o_ref):
    x = x_ref[0]
    i = lax.bitcast_convert_type(x, jnp.int32)
    m = i ^ ((i >> 31) & jnp.int32(0x7FFFFFFF))

    kf = jnp.float32(_K)

    cmax = jnp.max(m.reshape(_ROWS, _CHUNKS, _M // _CHUNKS), axis=-1)
    lo0 = jnp.min(cmax, axis=-1, keepdims=True)
    hi0 = jnp.max(cmax, axis=-1, keepdims=True) + jnp.int32(1)
    cnt0 = jnp.sum((m >= lo0).astype(jnp.float32), axis=-1, keepdims=True)

    def cond(carry):
        it, lo, hi, cnt = carry
        return (it < 34) & jnp.logical_not(
            jnp.all((cnt == kf) | (hi - lo == 1)))

    def body(carry):
        it, lo, hi, cnt = carry
        mid = (lo >> 1) + (hi >> 1) + (lo & hi & 1)
        c = jnp.sum((m >= mid).astype(jnp.float32), axis=-1, keepdims=True)
        take = c >= kf
        lo = jnp.where(take, mid, lo)
        cnt = jnp.where(take, c, cnt)
        hi = jnp.where(take, hi, mid)
        return it + 1, lo, hi, cnt

    _, p, _, cnt = lax.while_loop(cond, body, (jnp.int32(0), lo0, hi0, cnt0))

    all_resolved = jnp.all(cnt == kf)
    eq = m == p
    cnt_eq = jnp.sum(eq.astype(jnp.float32), axis=-1, keepdims=True)
    r = kf - (cnt - cnt_eq)
    idx = lax.broadcasted_iota(jnp.int32, (_ROWS, _M), 1)

    def cond2(carry):
        b2, _ = carry
        return (b2 >= 0) & jnp.logical_not(all_resolved)

    def body2(carry):
        b2, p2 = carry
        t2 = p2 | (jnp.int32(1) << b2)
        f = jnp.sum((eq & (idx < t2)).astype(jnp.float32), axis=-1,
                    keepdims=True)
        return b2 - 1, jnp.where(f < r, t2, p2)

    _, p2 = lax.while_loop(cond2, body2,
                           (jnp.int32(14), jnp.zeros((_ROWS, 1), jnp.int32)))
    j = jnp.where(cnt == kf, jnp.int32(_M - 1), p2)
    mask = (m > p) | (eq & (idx <= j))

    kept = jnp.where(mask, x, jnp.float32(0.0))
    s = jnp.sum(kept, axis=-1, keepdims=True) + jnp.float32(1e-12)
    o_ref[0] = kept * (jnp.float32(1.0) / s)


def kernel(logits):
    C, L, M = logits.shape
    grid = (C * L) // _ROWS
    x = logits.reshape(grid, _ROWS, M)
    out = pl.pallas_call(
        _topk_mask_body,
        grid=(grid,),
        in_specs=[pl.BlockSpec((1, _ROWS, M), lambda g: (g, 0, 0))],
        out_specs=pl.BlockSpec((1, _ROWS, M), lambda g: (g, 0, 0)),
        out_shape=jax.ShapeDtypeStruct((grid, _ROWS, M), jnp.float32),
    )(x)
    return out.reshape(C, L, M)


# speedup vs baseline: 51.7368x; 1.1414x over previous
import functools

import jax
import jax.numpy as jnp
from jax import lax
from jax.experimental import pallas as pl

_K = 64
_ROWS = 64
_M = 32768
_CHUNKS = 64


def _topk_mask_body(x_ref, o_ref):
    x = x_ref[0]
    i = lax.bitcast_convert_type(x, jnp.int32)
    m = i ^ ((i >> 31) & jnp.int32(0x7FFFFFFF))

    kf = jnp.float32(_K)

    cmax = jnp.max(m.reshape(_ROWS, _CHUNKS, _M // _CHUNKS), axis=-1)
    lo0 = jnp.min(cmax, axis=-1, keepdims=True)
    hi0 = jnp.max(cmax, axis=-1, keepdims=True) + jnp.int32(1)
    cnt0 = jnp.sum((m >= lo0).astype(jnp.float32), axis=-1, keepdims=True)

    def cond(carry):
        it, lo, hi, cnt = carry
        return (it < 34) & jnp.logical_not(
            jnp.all((cnt == kf) | (hi - lo == 1)))

    def body(carry):
        it, lo, hi, cnt = carry
        mid = (lo >> 1) + (hi >> 1) + (lo & hi & 1)
        c = jnp.sum((m >= mid).astype(jnp.float32), axis=-1, keepdims=True)
        take = c >= kf
        lo = jnp.where(take, mid, lo)
        cnt = jnp.where(take, c, cnt)
        hi = jnp.where(take, hi, mid)
        return it + 1, lo, hi, cnt

    _, p, _, cnt = lax.while_loop(cond, body, (jnp.int32(0), lo0, hi0, cnt0))

    all_resolved = jnp.all(cnt == kf)
    eq = m == p
    cnt_eq = jnp.sum(eq.astype(jnp.float32), axis=-1, keepdims=True)
    r = kf - (cnt - cnt_eq)
    idx = lax.broadcasted_iota(jnp.int32, (_ROWS, _M), 1)

    def cond2(carry):
        b2, _ = carry
        return (b2 >= 0) & jnp.logical_not(all_resolved)

    def body2(carry):
        b2, p2 = carry
        t2 = p2 | (jnp.int32(1) << b2)
        f = jnp.sum((eq & (idx < t2)).astype(jnp.float32), axis=-1,
                    keepdims=True)
        return b2 - 1, jnp.where(f < r, t2, p2)

    _, p2 = lax.while_loop(cond2, body2,
                           (jnp.int32(14), jnp.zeros((_ROWS, 1), jnp.int32)))
    j = jnp.where(cnt == kf, jnp.int32(_M - 1), p2)
    mask = (m > p) | (eq & (idx <= j))

    kept = jnp.where(mask, x, jnp.float32(0.0))
    s = jnp.sum(kept, axis=-1, keepdims=True) + jnp.float32(1e-12)
    o_ref[0] = kept * (jnp.float32(1.0) / s)


def kernel(logits):
    C, L, M = logits.shape
    grid = (C * L) // _ROWS
    x = logits.reshape(grid, _ROWS, M)
    out = pl.pallas_call(
        _topk_mask_body,
        grid=(grid,),
        in_specs=[pl.BlockSpec((1, _ROWS, M), lambda g: (g, 0, 0))],
        out_specs=pl.BlockSpec((1, _ROWS, M), lambda g: (g, 0, 0)),
        out_shape=jax.ShapeDtypeStruct((grid, _ROWS, M), jnp.float32),
    )(x)
    return out.reshape(C, L, M)
